# initial kernel scaffold (unmeasured)
import jax
import jax.numpy as jnp
from jax import lax
from jax.experimental import pallas as pl
from jax.experimental.pallas import tpu as pltpu


def kernel(
    x,
):
    def body(*refs):
        pass

    out_shape = jax.ShapeDtypeStruct(..., jnp.float32)
    return pl.pallas_call(body, out_shape=out_shape)(...)


# baseline (device time: 13884 ns/iter reference)
import jax
import jax.numpy as jnp
from jax import lax
from jax.experimental import pallas as pl
from jax.experimental.pallas import tpu as pltpu

N_DEV = 16
LOG2 = 4


def kernel(x):
    m, n = x.shape

    def body(x_ref, out_ref, send_buf, recv_buf, send_sems, recv_sems):
        my = lax.axis_index("i")

        barrier = pltpu.get_barrier_semaphore()
        for r in range(LOG2):
            partner = my ^ (1 << r)
            pl.semaphore_signal(
                barrier, inc=1,
                device_id=(partner,), device_id_type=pl.DeviceIdType.MESH,
            )
        pl.semaphore_wait(barrier, LOG2)

        acc = jnp.sum(x_ref[...], axis=0, keepdims=True)

        for r in range(LOG2):
            partner = my ^ (1 << r)
            send_buf[r, :, :] = acc
            rdma = pltpu.make_async_remote_copy(
                src_ref=send_buf.at[r],
                dst_ref=recv_buf.at[r],
                send_sem=send_sems.at[r],
                recv_sem=recv_sems.at[r],
                device_id=(partner,),
                device_id_type=pl.DeviceIdType.MESH,
            )
            rdma.start()
            rdma.wait()
            acc = acc + recv_buf[r, :, :]

        out_ref[...] = acc

    return pl.pallas_call(
        body,
        out_shape=jax.ShapeDtypeStruct((1, n), x.dtype),
        in_specs=[pl.BlockSpec(memory_space=pltpu.VMEM)],
        out_specs=pl.BlockSpec(memory_space=pltpu.VMEM),
        scratch_shapes=[
            pltpu.VMEM((LOG2, 1, n), x.dtype),
            pltpu.VMEM((LOG2, 1, n), x.dtype),
            pltpu.SemaphoreType.DMA((LOG2,)),
            pltpu.SemaphoreType.DMA((LOG2,)),
        ],
        compiler_params=pltpu.CompilerParams(collective_id=0),
    )(x)


# device time: 9860 ns/iter; 1.4081x vs baseline; 1.4081x over previous
import jax
import jax.numpy as jnp
from jax import lax
from jax.experimental import pallas as pl
from jax.experimental.pallas import tpu as pltpu

N_DEV = 16


def kernel(x):
    m, n = x.shape

    def body(x_ref, out_ref, send_buf, gather_buf, send_sems, recv_sems):
        my = lax.axis_index("i")

        barrier = pltpu.get_barrier_semaphore()
        for d in range(1, N_DEV):
            tgt = lax.rem(my + d, N_DEV)
            pl.semaphore_signal(
                barrier, inc=1,
                device_id=(tgt,), device_id_type=pl.DeviceIdType.MESH,
            )
        pl.semaphore_wait(barrier, N_DEV - 1)

        local = jnp.sum(x_ref[...], axis=0, keepdims=True)
        send_buf[...] = local
        gather_buf[my, :, :] = local

        rdmas = []
        for d in range(1, N_DEV):
            tgt = lax.rem(my + d, N_DEV)
            rdma = pltpu.make_async_remote_copy(
                src_ref=send_buf,
                dst_ref=gather_buf.at[my],
                send_sem=send_sems.at[d],
                recv_sem=recv_sems.at[my],
                device_id=(tgt,),
                device_id_type=pl.DeviceIdType.MESH,
            )
            rdma.start()
            rdmas.append(rdma)

        for d in range(1, N_DEV):
            src = lax.rem(my + d, N_DEV)
            recv = pltpu.make_async_remote_copy(
                src_ref=send_buf,
                dst_ref=gather_buf.at[src],
                send_sem=send_sems.at[0],
                recv_sem=recv_sems.at[src],
                device_id=(src,),
                device_id_type=pl.DeviceIdType.MESH,
            )
            recv.wait_recv()

        out_ref[...] = jnp.sum(gather_buf[...], axis=0)

        for rdma in rdmas:
            rdma.wait_send()

    return pl.pallas_call(
        body,
        out_shape=jax.ShapeDtypeStruct((1, n), x.dtype),
        in_specs=[pl.BlockSpec(memory_space=pltpu.VMEM)],
        out_specs=pl.BlockSpec(memory_space=pltpu.VMEM),
        scratch_shapes=[
            pltpu.VMEM((1, n), x.dtype),
            pltpu.VMEM((N_DEV, 1, n), x.dtype),
            pltpu.SemaphoreType.DMA((N_DEV,)),
            pltpu.SemaphoreType.DMA((N_DEV,)),
        ],
        compiler_params=pltpu.CompilerParams(collective_id=0),
    )(x)


# device time: 9446 ns/iter; 1.4698x vs baseline; 1.0438x over previous
import jax
import jax.numpy as jnp
from jax import lax
from jax.experimental import pallas as pl
from jax.experimental.pallas import tpu as pltpu

N_DEV = 16


def kernel(x):
    m, n = x.shape

    def body(x_ref, out_ref, send_buf, gather_buf, send_sems, recv_sems):
        my = lax.axis_index("i")

        barrier = pltpu.get_barrier_semaphore()
        for d in range(1, N_DEV):
            tgt = lax.rem(my + d, N_DEV)
            pl.semaphore_signal(
                barrier, inc=1,
                device_id=(tgt,), device_id_type=pl.DeviceIdType.MESH,
            )

        local = jnp.sum(x_ref[...], axis=0, keepdims=True)
        send_buf[...] = local
        gather_buf[my, :, :] = local

        pl.semaphore_wait(barrier, N_DEV - 1)

        rdmas = []
        for d in range(1, N_DEV):
            tgt = lax.rem(my + d, N_DEV)
            rdma = pltpu.make_async_remote_copy(
                src_ref=send_buf,
                dst_ref=gather_buf.at[my],
                send_sem=send_sems.at[d],
                recv_sem=recv_sems.at[my],
                device_id=(tgt,),
                device_id_type=pl.DeviceIdType.MESH,
            )
            rdma.start()
            rdmas.append(rdma)

        for d in range(1, N_DEV):
            src = lax.rem(my + d, N_DEV)
            recv = pltpu.make_async_remote_copy(
                src_ref=send_buf,
                dst_ref=gather_buf.at[src],
                send_sem=send_sems.at[0],
                recv_sem=recv_sems.at[src],
                device_id=(src,),
                device_id_type=pl.DeviceIdType.MESH,
            )
            recv.wait_recv()

        out_ref[...] = jnp.sum(gather_buf[...], axis=0)

        for rdma in rdmas:
            rdma.wait_send()

        for d in range(1, N_DEV):
            tgt = lax.rem(my + d, N_DEV)
            pl.semaphore_signal(
                barrier, inc=1,
                device_id=(tgt,), device_id_type=pl.DeviceIdType.MESH,
            )

    return pl.pallas_call(
        body,
        out_shape=jax.ShapeDtypeStruct((1, n), x.dtype),
        in_specs=[pl.BlockSpec(memory_space=pltpu.VMEM)],
        out_specs=pl.BlockSpec(memory_space=pltpu.VMEM),
        scratch_shapes=[
            pltpu.VMEM((1, n), x.dtype),
            pltpu.VMEM((N_DEV, 1, n), x.dtype),
            pltpu.SemaphoreType.DMA((N_DEV,)),
            pltpu.SemaphoreType.DMA((N_DEV,)),
        ],
        compiler_params=pltpu.CompilerParams(collective_id=0),
    )(x)
